# baseline (speedup 1.0000x reference)
import jax
import jax.numpy as jnp
from jax import lax
from jax.experimental import pallas as pl
from jax.experimental.pallas import tpu as pltpu
from jax.experimental.pallas import tpu_sc as plsc

_N = 131072
_ZROWS = 86
_D = 256

_NC = 2
_NS = 16
_NW = _NC * _NS
_BPW = _N // _NW
_C = 64
_NCHUNK = _BPW // _C
_NBUF = 4


def _table_body(ec_ref, w_ref, zt_ref, out_ref):
    out_ref[...] = (
        jnp.dot(ec_ref[...], w_ref[...], preferred_element_type=jnp.float32)
        + zt_ref[...]
    )


def _fused_table(ec86, w, zt):
    return pl.pallas_call(
        _table_body,
        out_shape=jax.ShapeDtypeStruct((_ZROWS, _D), jnp.float32),
    )(ec86, w, zt)


def _gather_body(table_hbm, idx_hbm, out_hbm,
                 idx_v, bufs, gsems, osems):
    cid = lax.axis_index("c")
    sid = lax.axis_index("s")
    wid = sid * _NC + cid
    base = wid * _BPW

    pltpu.sync_copy(idx_hbm.at[pl.ds(base, _BPW)], idx_v)

    def start_gather(gi, b):
        pltpu.async_copy(table_hbm.at[idx_v.at[pl.ds(gi * _C, _C)]],
                         bufs[b], gsems[b])

    def wait_gather(b):
        pltpu.make_async_copy(out_hbm.at[pl.ds(base, _C)],
                              bufs[b], gsems[b]).wait()

    def start_scatter(gi, b):
        pltpu.async_copy(bufs[b], out_hbm.at[pl.ds(base + gi * _C, _C)],
                         osems[b])

    def wait_scatter(b):
        pltpu.make_async_copy(bufs[b], out_hbm.at[pl.ds(base, _C)],
                              osems[b]).wait()

    start_gather(0, 0)
    start_gather(1, 1)

    @pl.loop(0, _NCHUNK, step=_NBUF)
    def _chunks(g):
        for u in range(_NBUF):
            gi = g + u
            b = u
            bn = (u + 2) % _NBUF
            wait_gather(b)
            start_scatter(gi, b)
            nxt = gi + 2

            @pl.when(nxt < _NCHUNK)
            def _():
                @pl.when(nxt >= _NBUF)
                def _():
                    wait_scatter(bn)
                start_gather(nxt, bn)

    for k in range(_NBUF, 0, -1):
        wait_scatter((_NCHUNK - k) % _NBUF)


def kernel(z, elec_config, m_weight, z_table):
    zi = z.astype(jnp.int32)
    table = _fused_table(elec_config[:_ZROWS], m_weight, z_table)
    mesh = plsc.VectorSubcoreMesh(core_axis_name="c", subcore_axis_name="s",
                                  num_cores=_NC, num_subcores=_NS)
    gather = pl.kernel(
        _gather_body,
        out_type=jax.ShapeDtypeStruct((_N, _D), jnp.float32),
        mesh=mesh,
        scratch_types=[
            pltpu.VMEM((_BPW,), jnp.int32),
            [pltpu.VMEM((_C, _D), jnp.float32) for _ in range(_NBUF)],
            [pltpu.SemaphoreType.DMA for _ in range(_NBUF)],
            [pltpu.SemaphoreType.DMA for _ in range(_NBUF)],
        ],
    )
    return gather(table, zi)


# speedup vs baseline: 2.1964x; 2.1964x over previous
import jax
import jax.numpy as jnp
from jax import lax
from jax.experimental import pallas as pl
from jax.experimental.pallas import tpu as pltpu
from jax.experimental.pallas import tpu_sc as plsc

_N = 131072
_ZROWS = 86
_D = 256

_NC = 2
_NS = 16
_NW = _NC * _NS
_BPW = _N // _NW
_C = 64
_NCHUNK = _BPW // _C
_NBUF = 4


def _prep_body(ec_ref, w_ref, zt_ref, z_ref, tab_ref, idx_ref):
    t = (jnp.dot(ec_ref[...], w_ref[...], preferred_element_type=jnp.float32)
         + zt_ref[...])
    tab_ref[...] = jnp.broadcast_to(t[None], (_NW, _ZROWS, _D))
    shift = jax.lax.broadcasted_iota(jnp.int32, (_NW, _BPW), 0) * _ZROWS
    idx_ref[...] = z_ref[...] + shift


def _prep(ec86, w, zt, z2d):
    return pl.pallas_call(
        _prep_body,
        out_shape=(
            jax.ShapeDtypeStruct((_NW, _ZROWS, _D), jnp.float32),
            jax.ShapeDtypeStruct((_NW, _BPW), jnp.int32),
        ),
    )(ec86, w, zt, z2d)


def _gather_body(table_hbm, idx_hbm, out_hbm,
                 idx_v, bufs, gsems, osems):
    cid = lax.axis_index("c")
    sid = lax.axis_index("s")
    wid = sid * _NC + cid
    base = wid * _BPW

    pltpu.sync_copy(idx_hbm.at[pl.ds(base, _BPW)], idx_v)

    def start_gather(gi, b):
        pltpu.async_copy(table_hbm.at[idx_v.at[pl.ds(gi * _C, _C)]],
                         bufs[b], gsems[b])

    def wait_gather(b):
        pltpu.make_async_copy(out_hbm.at[pl.ds(base, _C)],
                              bufs[b], gsems[b]).wait()

    def start_scatter(gi, b):
        pltpu.async_copy(bufs[b], out_hbm.at[pl.ds(base + gi * _C, _C)],
                         osems[b])

    def wait_scatter(b):
        pltpu.make_async_copy(bufs[b], out_hbm.at[pl.ds(base, _C)],
                              osems[b]).wait()

    start_gather(0, 0)
    start_gather(1, 1)

    @pl.loop(0, _NCHUNK, step=_NBUF)
    def _chunks(g):
        for u in range(_NBUF):
            gi = g + u
            b = u
            bn = (u + 2) % _NBUF
            wait_gather(b)
            start_scatter(gi, b)
            nxt = gi + 2

            @pl.when(nxt < _NCHUNK)
            def _():
                @pl.when(nxt >= _NBUF)
                def _():
                    wait_scatter(bn)
                start_gather(nxt, bn)

    for k in range(_NBUF, 0, -1):
        wait_scatter((_NCHUNK - k) % _NBUF)


def kernel(z, elec_config, m_weight, z_table):
    zi = z.astype(jnp.int32).reshape(_NW, _BPW)
    tab, idx = _prep(elec_config[:_ZROWS], m_weight, z_table, zi)
    tab = tab.reshape(_NW * _ZROWS, _D)
    idx = idx.reshape(_N)
    mesh = plsc.VectorSubcoreMesh(core_axis_name="c", subcore_axis_name="s",
                                  num_cores=_NC, num_subcores=_NS)
    gather = pl.kernel(
        _gather_body,
        out_type=jax.ShapeDtypeStruct((_N, _D), jnp.float32),
        mesh=mesh,
        scratch_types=[
            pltpu.VMEM((_BPW,), jnp.int32),
            [pltpu.VMEM((_C, _D), jnp.float32) for _ in range(_NBUF)],
            [pltpu.SemaphoreType.DMA for _ in range(_NBUF)],
            [pltpu.SemaphoreType.DMA for _ in range(_NBUF)],
        ],
    )
    return gather(tab, idx)


# speedup vs baseline: 2.2025x; 1.0028x over previous
import jax
import jax.numpy as jnp
from jax import lax
from jax.experimental import pallas as pl
from jax.experimental.pallas import tpu as pltpu
from jax.experimental.pallas import tpu_sc as plsc

_N = 131072
_ZROWS = 86
_D = 256

_NC = 2
_NS = 16
_NW = _NC * _NS
_BPW = _N // _NW
_C = 32
_NCHUNK = _BPW // _C
_NBUF = 8
_LOOK = 4


def _prep_body(ec_ref, w_ref, zt_ref, z_ref, tab_ref, idx_ref):
    t = (jnp.dot(ec_ref[...], w_ref[...], preferred_element_type=jnp.float32)
         + zt_ref[...])
    tab_ref[...] = jnp.broadcast_to(t[None], (_NW, _ZROWS, _D))
    shift = jax.lax.broadcasted_iota(jnp.int32, (_NW, _BPW), 0) * _ZROWS
    idx_ref[...] = z_ref[...] + shift


def _prep(ec86, w, zt, z2d):
    return pl.pallas_call(
        _prep_body,
        out_shape=(
            jax.ShapeDtypeStruct((_NW, _ZROWS, _D), jnp.float32),
            jax.ShapeDtypeStruct((_NW, _BPW), jnp.int32),
        ),
    )(ec86, w, zt, z2d)


def _gather_body(table_hbm, idx_hbm, out_hbm,
                 idx_v, bufs, gsems, osems):
    cid = lax.axis_index("c")
    sid = lax.axis_index("s")
    wid = sid * _NC + cid
    base = wid * _BPW

    pltpu.sync_copy(idx_hbm.at[pl.ds(base, _BPW)], idx_v)

    def start_gather(gi, b):
        pltpu.async_copy(table_hbm.at[idx_v.at[pl.ds(gi * _C, _C)]],
                         bufs[b], gsems[b])

    def wait_gather(b):
        pltpu.make_async_copy(out_hbm.at[pl.ds(base, _C)],
                              bufs[b], gsems[b]).wait()

    def start_scatter(gi, b):
        pltpu.async_copy(bufs[b], out_hbm.at[pl.ds(base + gi * _C, _C)],
                         osems[b])

    def wait_scatter(b):
        pltpu.make_async_copy(bufs[b], out_hbm.at[pl.ds(base, _C)],
                              osems[b]).wait()

    for k in range(_LOOK):
        start_gather(k, k)

    @pl.loop(0, _NCHUNK, step=_NBUF)
    def _chunks(g):
        for u in range(_NBUF):
            gi = g + u
            b = u
            bn = (u + _LOOK) % _NBUF
            wait_gather(b)
            start_scatter(gi, b)
            nxt = gi + _LOOK

            @pl.when(nxt < _NCHUNK)
            def _():
                @pl.when(nxt >= _NBUF)
                def _():
                    wait_scatter(bn)
                start_gather(nxt, bn)

    for k in range(_NBUF, 0, -1):
        wait_scatter((_NCHUNK - k) % _NBUF)


def kernel(z, elec_config, m_weight, z_table):
    zi = z.astype(jnp.int32).reshape(_NW, _BPW)
    tab, idx = _prep(elec_config[:_ZROWS], m_weight, z_table, zi)
    tab = tab.reshape(_NW * _ZROWS, _D)
    idx = idx.reshape(_N)
    mesh = plsc.VectorSubcoreMesh(core_axis_name="c", subcore_axis_name="s",
                                  num_cores=_NC, num_subcores=_NS)
    gather = pl.kernel(
        _gather_body,
        out_type=jax.ShapeDtypeStruct((_N, _D), jnp.float32),
        mesh=mesh,
        scratch_types=[
            pltpu.VMEM((_BPW,), jnp.int32),
            [pltpu.VMEM((_C, _D), jnp.float32) for _ in range(_NBUF)],
            [pltpu.SemaphoreType.DMA for _ in range(_NBUF)],
            [pltpu.SemaphoreType.DMA for _ in range(_NBUF)],
        ],
    )
    return gather(tab, idx)
